# baseline (speedup 1.0000x reference)
import jax
import jax.numpy as jnp
from jax.experimental import pallas as pl
from jax.experimental.pallas import tpu as pltpu

D_MODEL = 4096
NUM_E = 64
N_TOK = 4 * 4096
TOK_BLK = 1024
GRID = N_TOK // TOK_BLK


def _router_body(x_ref, wt_ref, b_ref, top1_ref, prob_ref, stats_ref, aux_ref):
    i = pl.program_id(0)
    logits = jnp.dot(x_ref[...], wt_ref[...],
                     preferred_element_type=jnp.float32) + b_ref[...]
    m = jnp.max(logits, axis=-1, keepdims=True)
    e = jnp.exp(logits - m)
    s = jnp.sum(e, axis=-1, keepdims=True)
    rs = 1.0 / s
    top1 = jnp.argmax(logits, axis=-1).astype(jnp.int32)
    top1_ref[0, 0, :] = top1
    prob_ref[0, 0, :] = rs[:, 0]

    imp_part = jnp.sum(e * rs, axis=0)
    iota = jax.lax.broadcasted_iota(jnp.int32, (TOK_BLK, NUM_E), 1)
    cnt_part = jnp.sum((iota == top1[:, None]).astype(jnp.float32), axis=0)
    part = jnp.concatenate([imp_part[None, :], cnt_part[None, :]], axis=0)

    @pl.when(i == 0)
    def _init():
        stats_ref[...] = jnp.zeros_like(stats_ref)

    stats_ref[...] += part

    @pl.when(i == GRID - 1)
    def _finish():
        st = stats_ref[...]
        aux_ref[...] = (NUM_E / (N_TOK * N_TOK)) * jnp.sum(
            st[0:1, :] * st[1:2, :], axis=1, keepdims=True)


def kernel(x, W, b):
    xf = x.reshape(N_TOK, D_MODEL)
    wt = W.T
    b2 = b.reshape(1, NUM_E)
    top1, prob, _, aux = pl.pallas_call(
        _router_body,
        grid=(GRID,),
        in_specs=[
            pl.BlockSpec((TOK_BLK, D_MODEL), lambda i: (i, 0)),
            pl.BlockSpec((D_MODEL, NUM_E), lambda i: (0, 0)),
            pl.BlockSpec((1, NUM_E), lambda i: (0, 0)),
        ],
        out_specs=[
            pl.BlockSpec((1, 1, TOK_BLK), lambda i: (i, 0, 0)),
            pl.BlockSpec((1, 1, TOK_BLK), lambda i: (i, 0, 0)),
            pl.BlockSpec((2, NUM_E), lambda i: (0, 0)),
            pl.BlockSpec((1, 1), lambda i: (0, 0)),
        ],
        out_shape=[
            jax.ShapeDtypeStruct((GRID, 1, TOK_BLK), jnp.int32),
            jax.ShapeDtypeStruct((GRID, 1, TOK_BLK), jnp.float32),
            jax.ShapeDtypeStruct((2, NUM_E), jnp.float32),
            jax.ShapeDtypeStruct((1, 1), jnp.float32),
        ],
        compiler_params=pltpu.CompilerParams(
            dimension_semantics=("arbitrary",),
        ),
    )(xf, wt, b2)
    return (top1.reshape(x.shape[0], x.shape[1]),
            prob.reshape(x.shape[0], x.shape[1]),
            aux.reshape(()))
